# baseline (speedup 1.0000x reference)
import jax
import jax.numpy as jnp
from jax.experimental import pallas as pl
from jax.experimental.pallas import tpu as pltpu

_TILE = 8192


def _fused_kernel(x_ref, w_ref, b_ref, out_ref):
    z = jnp.dot(x_ref[...], w_ref[...], preferred_element_type=jnp.float32)
    out_ref[...] = jnp.maximum(z + b_ref[...], 0.0)


def kernel(x, beta, w, b):
    n, d = x.shape
    e = w.shape[1]
    w_eff = beta * w
    b_row = b.reshape(1, e)

    p = 128 // d if (d < 128 and 128 % d == 0) else 1
    if p > 1:
        b_p = jnp.tile(b_row, (1, p))
        w_p = jax.scipy.linalg.block_diag(*([w_eff] * p))
        dp, ep = p * d, p * e
    else:
        b_p, w_p, dp, ep = b_row, w_eff, d, e

    rows = n // p
    tile = min(_TILE, ((rows + 7) // 8) * 8)
    rows_pad = ((rows + tile - 1) // tile) * tile
    xp = x.reshape(rows, dp)
    if rows_pad != rows:
        xp = jnp.pad(xp, ((0, rows_pad - rows), (0, 0)))

    out = pl.pallas_call(
        _fused_kernel,
        out_shape=jax.ShapeDtypeStruct((rows_pad, ep), jnp.float32),
        grid=(rows_pad // tile,),
        in_specs=[
            pl.BlockSpec((tile, dp), lambda i: (i, 0)),
            pl.BlockSpec((dp, ep), lambda i: (0, 0)),
            pl.BlockSpec((1, ep), lambda i: (0, 0)),
        ],
        out_specs=pl.BlockSpec((tile, ep), lambda i: (i, 0)),
        compiler_params=pltpu.CompilerParams(
            dimension_semantics=("parallel",)),
    )(xp, w_p, b_p)
    return out[:rows].reshape(n, e)
